# baseline (speedup 1.0000x reference)
import jax
import jax.numpy as jnp
from jax import lax
from jax.experimental import pallas as pl
from jax.experimental.pallas import tpu as pltpu
from jax.experimental.pallas import tpu_sc as plsc

NC = 2
NS = 16
NW = NC * NS

V = 1000000
EMB = 32
B = 16384
L = 200

BPW = B // NW
CB = 8
NCHUNK = BPW // CB
LANES = 16


def _lookup_body(ids_hbm, table_hbm, out_hbm,
                 idx0, idx1, rows0, rows1, out_v,
                 gsem0, gsem1, isem0, isem1):
    wid = lax.axis_index("s") * NC + lax.axis_index("c")
    base = wid * BPW
    scale = jnp.float32(1.0 / L)
    z = jnp.zeros((LANES,), jnp.float32)

    def idx_start(c):
        return (base + c * CB) * L

    pltpu.sync_copy(ids_hbm.at[pl.ds(idx_start(0), CB * L)], idx0)
    pltpu.async_copy(table_hbm.at[idx0], rows0, gsem0)
    pltpu.async_copy(ids_hbm.at[pl.ds(idx_start(1), CB * L)], idx1, isem1)

    bufs = ((idx0, rows0, gsem0), (idx1, rows1, gsem1))
    isems = (isem0, isem1)

    def outer(c2, carry):
        for b in range(2):
            c = c2 + b
            idx_c, rows_c, gsem_c = bufs[b]
            idx_n, rows_n, gsem_n = bufs[1 - b]
            pltpu.make_async_copy(table_hbm.at[idx_c], rows_c, gsem_c).wait()

            @pl.when(c + 1 < NCHUNK)
            def _():
                pltpu.make_async_copy(
                    ids_hbm.at[pl.ds(idx_start(c + 1), CB * L)],
                    idx_n, isems[1 - b]).wait()
                pltpu.async_copy(table_hbm.at[idx_n], rows_n, gsem_n)

            @pl.when(c + 2 < NCHUNK)
            def _():
                pltpu.async_copy(
                    ids_hbm.at[pl.ds(idx_start(c + 2), CB * L)],
                    idx_c, isems[b])

            for j in range(CB):
                off = j * L

                def red(i, acc):
                    a0, a1 = acc
                    return (a0 + rows_c[off + i, pl.ds(0, LANES)],
                            a1 + rows_c[off + i, pl.ds(LANES, LANES)])

                a0, a1 = lax.fori_loop(0, L, red, (z, z), unroll=8)
                row = c * CB + j
                out_v[row, pl.ds(0, LANES)] = a0 * scale
                out_v[row, pl.ds(LANES, LANES)] = a1 * scale
        return carry

    lax.fori_loop(0, NCHUNK // 2, lambda i, u: outer(i * 2, u), 0)
    pltpu.sync_copy(out_v, out_hbm.at[pl.ds(base, BPW)])


_MESH = dict(core_axis_name="c", subcore_axis_name="s",
             num_cores=NC, num_subcores=NS)


@jax.jit
def kernel(token_ids, emb_weight):
    lookup = pl.kernel(
        _lookup_body,
        out_type=jax.ShapeDtypeStruct((B, EMB), jnp.float32),
        mesh=plsc.VectorSubcoreMesh(**_MESH),
        scratch_types=[
            pltpu.VMEM((CB * L,), jnp.int32),
            pltpu.VMEM((CB * L,), jnp.int32),
            pltpu.VMEM((CB * L, EMB), jnp.float32),
            pltpu.VMEM((CB * L, EMB), jnp.float32),
            pltpu.VMEM((BPW, EMB), jnp.float32),
            pltpu.SemaphoreType.DMA,
            pltpu.SemaphoreType.DMA,
            pltpu.SemaphoreType.DMA,
            pltpu.SemaphoreType.DMA,
        ],
        compiler_params=pltpu.CompilerParams(use_tc_tiling_on_sc=False),
    )
    ids_flat = token_ids.reshape(B * L).astype(jnp.int32)
    return lookup(ids_flat, emb_weight)
